# Initial kernel scaffold; baseline (speedup 1.0000x reference)
import jax
import jax.numpy as jnp
from jax.experimental import pallas as pl


def kernel(cycle_curve_data, logits, moe_masks, W_in, W_out, b_out):
    raise NotImplementedError("write your pallas kernel here")


# speedup vs baseline: 2.0481x; 2.0481x over previous
import functools

import jax
import jax.numpy as jnp
from jax.experimental import pallas as pl
from jax.experimental.pallas import tpu as pltpu

E = 8
TOPK = 2
EPS = 1e-09
TB = 256


def _moe_block(x_ref, lg_ref, mk_ref, win_ref, wout_ref, b_ref, o_ref):
    lg = lg_ref[...]
    mk = mk_ref[...]
    m = jnp.max(lg, axis=1, keepdims=True)
    ex = jnp.exp(lg - m)
    g = ex / jnp.sum(ex, axis=1, keepdims=True)
    g = jnp.where(mk == 1, g, 0.0)
    eidx = jax.lax.broadcasted_iota(jnp.int32, g.shape, 1)
    m1 = jnp.max(g, axis=1, keepdims=True)
    i1 = jnp.min(jnp.where(g == m1, eidx, E), axis=1, keepdims=True)
    g2 = jnp.where(eidx == i1, -1.0, g)
    m2 = jnp.max(g2, axis=1, keepdims=True)
    i2 = jnp.min(jnp.where(g2 == m2, eidx, E), axis=1, keepdims=True)
    sel = (eidx == i1) | (eidx == i2)
    denom = m1 + m2 + EPS
    gf = jnp.where(sel, g, 0.0) / denom

    xb = x_ref[...].astype(jnp.bfloat16)
    acc = jnp.dot(gf.astype(jnp.bfloat16), b_ref[...].astype(jnp.bfloat16),
                  preferred_element_type=jnp.float32)
    for e in range(E):
        h = jnp.dot(xb, win_ref[e], preferred_element_type=jnp.float32)
        h = h * 0.5 * (1.0 + jax.lax.erf(h * 0.7071067811865476))
        hb = (h * gf[:, e:e + 1]).astype(jnp.bfloat16)
        acc = acc + jnp.dot(hb, wout_ref[e], preferred_element_type=jnp.float32)
    o_ref[...] = acc.astype(jnp.bfloat16)


@functools.partial(jax.jit, static_argnames=())
def kernel(cycle_curve_data, logits, moe_masks, W_in, W_out, b_out):
    B, L, D = cycle_curve_data.shape
    FF = W_in.shape[2]
    x = cycle_curve_data.reshape(B, D)
    win = W_in.astype(jnp.bfloat16)
    wout = W_out.astype(jnp.bfloat16)
    grid = (B // TB,)
    out = pl.pallas_call(
        _moe_block,
        grid=grid,
        in_specs=[
            pl.BlockSpec((TB, D), lambda i: (i, 0)),
            pl.BlockSpec((TB, E), lambda i: (i, 0)),
            pl.BlockSpec((TB, E), lambda i: (i, 0)),
            pl.BlockSpec((E, D, FF), lambda i: (0, 0, 0)),
            pl.BlockSpec((E, FF, D), lambda i: (0, 0, 0)),
            pl.BlockSpec((E, D), lambda i: (0, 0)),
        ],
        out_specs=pl.BlockSpec((TB, D), lambda i: (i, 0)),
        out_shape=jax.ShapeDtypeStruct((B, D), jnp.bfloat16),
        compiler_params=pltpu.CompilerParams(
            dimension_semantics=("arbitrary",),
        ),
    )(x, logits, moe_masks, win, wout, b_out)
    return out.reshape(B, L, D)


# speedup vs baseline: 2.0876x; 1.0193x over previous
import functools

import jax
import jax.numpy as jnp
from jax.experimental import pallas as pl
from jax.experimental.pallas import tpu as pltpu

E = 8
TOPK = 2
EPS = 1e-09
TB = 2048


def _moe_block(x_ref, lg_ref, mk_ref, win_ref, wout_ref, b_ref, o_ref):
    lg = lg_ref[...]
    mk = mk_ref[...]
    m = jnp.max(lg, axis=1, keepdims=True)
    ex = jnp.exp(lg - m)
    g = ex / jnp.sum(ex, axis=1, keepdims=True)
    g = jnp.where(mk == 1, g, 0.0)
    eidx = jax.lax.broadcasted_iota(jnp.int32, g.shape, 1)
    m1 = jnp.max(g, axis=1, keepdims=True)
    i1 = jnp.min(jnp.where(g == m1, eidx, E), axis=1, keepdims=True)
    g2 = jnp.where(eidx == i1, -1.0, g)
    m2 = jnp.max(g2, axis=1, keepdims=True)
    i2 = jnp.min(jnp.where(g2 == m2, eidx, E), axis=1, keepdims=True)
    sel = (eidx == i1) | (eidx == i2)
    denom = m1 + m2 + EPS
    gf = jnp.where(sel, g, 0.0) / denom

    xb = x_ref[...].astype(jnp.bfloat16)
    acc = jnp.dot(gf.astype(jnp.bfloat16), b_ref[...].astype(jnp.bfloat16),
                  preferred_element_type=jnp.float32)
    for e in range(E):
        h = jnp.dot(xb, win_ref[e], preferred_element_type=jnp.float32)
        h = h * 0.5 * (1.0 + jax.lax.erf(h * 0.7071067811865476))
        hb = (h * gf[:, e:e + 1]).astype(jnp.bfloat16)
        acc = acc + jnp.dot(hb, wout_ref[e], preferred_element_type=jnp.float32)
    o_ref[...] = acc.astype(jnp.bfloat16)


@functools.partial(jax.jit, static_argnames=())
def kernel(cycle_curve_data, logits, moe_masks, W_in, W_out, b_out):
    B, L, D = cycle_curve_data.shape
    FF = W_in.shape[2]
    x = cycle_curve_data.reshape(B, D)
    win = W_in.astype(jnp.bfloat16)
    wout = W_out.astype(jnp.bfloat16)
    grid = (B // TB,)
    out = pl.pallas_call(
        _moe_block,
        grid=grid,
        in_specs=[
            pl.BlockSpec((TB, D), lambda i: (i, 0)),
            pl.BlockSpec((TB, E), lambda i: (i, 0)),
            pl.BlockSpec((TB, E), lambda i: (i, 0)),
            pl.BlockSpec((E, D, FF), lambda i: (0, 0, 0)),
            pl.BlockSpec((E, FF, D), lambda i: (0, 0, 0)),
            pl.BlockSpec((E, D), lambda i: (0, 0)),
        ],
        out_specs=pl.BlockSpec((TB, D), lambda i: (i, 0)),
        out_shape=jax.ShapeDtypeStruct((B, D), jnp.bfloat16),
        compiler_params=pltpu.CompilerParams(
            dimension_semantics=("arbitrary",),
        ),
    )(x, logits, moe_masks, win, wout, b_out)
    return out.reshape(B, L, D)
